# Initial kernel scaffold; baseline (speedup 1.0000x reference)
import jax
import jax.numpy as jnp
from jax.experimental import pallas as pl


def kernel(node_features, edge_features, Esrc, Etgt, batch, gc1_W, gc1_b, gc2_W, gc2_b, gc3_W, gc3_b, ee1_W1, ee1_b1, ee1_W2, ee1_b2, ee2_W1, ee2_b1, ee2_W2, ee2_b2, ee3_W1, ee3_b1, ee3_W2, ee3_b2):
    raise NotImplementedError("write your pallas kernel here")


# speedup vs baseline: 3.0730x; 3.0730x over previous
import functools

import jax
import jax.numpy as jnp
from jax import lax
from jax.experimental import pallas as pl
from jax.experimental.pallas import tpu as pltpu
from jax.experimental.pallas import tpu_sc as plsc

_N = 10000
_E = 320000
_D = 128
_DE = 16
_B = 64

_NC = 2
_NS = 16
_NW = _NC * _NS
_EPW = _E // _NW
_C = 80
_NCH = _EPW // _C
_STRIPE = 624
_TAIL = _N - _NS * _STRIPE
_ZR = 104

_EB = 3200
_NB = 2000

_f32 = jnp.float32


def _gates_body(ef_ref, w11, b11, w12, b12, w21, b21, w22, b22,
                w31, b31, w32, b32, o1, o2, o3):
    ef = ef_ref[...]
    for w1, b1, w2, b2, o in ((w11, b11, w12, b12, o1),
                              (w21, b21, w22, b22, o2),
                              (w31, b31, w32, b32, o3)):
        h = jnp.maximum(
            jnp.dot(ef, w1[...], preferred_element_type=_f32) + b1[...], 0.0)
        o[...] = jax.nn.sigmoid(
            jnp.dot(h, w2[...], preferred_element_type=_f32) + b2[...])


def _edge_gates(ef, params):
    full = lambda arr: pl.BlockSpec(arr.shape, lambda i: (0,) * arr.ndim)
    in_specs = [pl.BlockSpec((_EB, _DE), lambda i: (i, 0))]
    args = [ef]
    for w1, b1, w2, b2 in params:
        b1 = b1.reshape(1, _D)
        b2 = b2.reshape(1, _D)
        for a in (w1, b1, w2, b2):
            in_specs.append(full(a))
            args.append(a)
    out = pl.pallas_call(
        _gates_body,
        grid=(_E // _EB,),
        in_specs=in_specs,
        out_specs=[pl.BlockSpec((_EB, _D), lambda i: (i, 0))] * 3,
        out_shape=[jax.ShapeDtypeStruct((_E, _D), _f32)] * 3,
    )(*args)
    return out


def _lin_body(x_ref, w_ref, b_ref, o_ref):
    o_ref[...] = (jnp.dot(x_ref[...], w_ref[...], preferred_element_type=_f32)
                  + b_ref[...])


def _linear(x, w, b):
    return pl.pallas_call(
        _lin_body,
        grid=(_N // _NB,),
        in_specs=[pl.BlockSpec((_NB, _D), lambda i: (i, 0)),
                  pl.BlockSpec((_D, _D), lambda i: (0, 0)),
                  pl.BlockSpec((1, _D), lambda i: (0, 0))],
        out_specs=pl.BlockSpec((_NB, _D), lambda i: (i, 0)),
        out_shape=jax.ShapeDtypeStruct((_N, _D), _f32),
    )(x, w, b.reshape(1, _D))


def _lin2_body(p_ref, w_ref, b_ref, o_ref):
    x = jnp.maximum(p_ref[0] + p_ref[1], 0.0)
    o_ref[...] = (jnp.dot(x, w_ref[...], preferred_element_type=_f32)
                  + b_ref[...])


def _linear_from_partials(p, w, b):
    return pl.pallas_call(
        _lin2_body,
        grid=(_N // _NB,),
        in_specs=[pl.BlockSpec((2, _NB, _D), lambda i: (0, i, 0)),
                  pl.BlockSpec((_D, _D), lambda i: (0, 0)),
                  pl.BlockSpec((1, _D), lambda i: (0, 0))],
        out_specs=pl.BlockSpec((_NB, _D), lambda i: (i, 0)),
        out_shape=jax.ShapeDtypeStruct((_N, _D), _f32),
    )(p, w, b.reshape(1, _D))


def _pool_body(p_ref, batch_ref, o_ref):
    i = pl.program_id(0)

    @pl.when(i == 0)
    def _():
        o_ref[...] = jnp.zeros_like(o_ref)

    x = p_ref[0] + p_ref[1]
    seg = batch_ref[...]
    gids = lax.broadcasted_iota(jnp.int32, (1, _B), 1)
    onehot = (seg == gids).astype(_f32)
    o_ref[...] += lax.dot_general(
        onehot, x, (((0,), (0,)), ((), ())), preferred_element_type=_f32)


def _pool(p, batch):
    return pl.pallas_call(
        _pool_body,
        grid=(_N // _NB,),
        in_specs=[pl.BlockSpec((2, _NB, _D), lambda i: (0, i, 0)),
                  pl.BlockSpec((_NB, 1), lambda i: (i, 0))],
        out_specs=pl.BlockSpec((_B, _D), lambda i: (0, 0)),
        out_shape=jax.ShapeDtypeStruct((_B, _D), _f32),
    )(p, batch.reshape(_N, 1))


def _sc_body(sup_hbm, ef_hbm, esrc_hbm, etgt_hbm, out_hbm,
             esrc_v, etgt_v, rows_v, ef_v, zrow_v, acc_sh, gsem):
    c = lax.axis_index("c")
    s = lax.axis_index("s")
    wid = c * _NS + s

    zero16 = jnp.zeros((16,), _f32)

    @pl.loop(0, _ZR)
    def _zfill(i):
        for j in range(_D // 16):
            zrow_v[i, pl.ds(j * 16, 16)] = zero16

    @pl.loop(0, _STRIPE // _ZR)
    def _zcopy(k):
        pltpu.sync_copy(zrow_v, acc_sh.at[pl.ds(s * _STRIPE + k * _ZR, _ZR)])

    @pl.when(s == _NS - 1)
    def _ztail():
        pltpu.sync_copy(zrow_v.at[pl.ds(0, _TAIL)],
                        acc_sh.at[pl.ds(_NS * _STRIPE, _TAIL)])

    plsc.subcore_barrier()

    @pl.loop(0, _NCH)
    def _chunk(i):
        base = wid * _EPW + i * _C
        pltpu.sync_copy(esrc_hbm.at[pl.ds(base, _C)], esrc_v)
        pltpu.sync_copy(etgt_hbm.at[pl.ds(base, _C)], etgt_v)
        gather = pltpu.async_copy(sup_hbm.at[esrc_v], rows_v, gsem)
        pltpu.sync_copy(ef_hbm.at[pl.ds(base, _C)], ef_v)
        gather.wait()

        @pl.loop(0, _C)
        def _mul(r):
            for j in range(_D // 16):
                sl = pl.ds(j * 16, 16)
                rows_v[r, sl] = rows_v[r, sl] * ef_v[r, sl]

        pltpu.sync_copy(rows_v, acc_sh.at[etgt_v], add=True)

    plsc.subcore_barrier()

    off = s * _STRIPE
    pltpu.sync_copy(acc_sh.at[pl.ds(off, _STRIPE)],
                    out_hbm.at[c, pl.ds(off, _STRIPE)])

    @pl.when(s == _NS - 1)
    def _otail():
        pltpu.sync_copy(acc_sh.at[pl.ds(_NS * _STRIPE, _TAIL)],
                        out_hbm.at[c, pl.ds(_NS * _STRIPE, _TAIL)])


@functools.cache
def _make_sc_layer():
    return pl.kernel(
        _sc_body,
        out_type=jax.ShapeDtypeStruct((_NC, _N, _D), _f32),
        mesh=plsc.VectorSubcoreMesh(core_axis_name="c", subcore_axis_name="s",
                                    num_cores=_NC, num_subcores=_NS),
        scratch_types=[
            pltpu.VMEM((_C,), jnp.int32),
            pltpu.VMEM((_C,), jnp.int32),
            pltpu.VMEM((_C, _D), _f32),
            pltpu.VMEM((_C, _D), _f32),
            pltpu.VMEM((_ZR, _D), _f32),
            pltpu.VMEM_SHARED((_N, _D), _f32),
            pltpu.SemaphoreType.DMA,
        ],
    )


def _sc_layer(*args):
    return _make_sc_layer()(*args)


def kernel(node_features, edge_features, Esrc, Etgt, batch,
           gc1_W, gc1_b, gc2_W, gc2_b, gc3_W, gc3_b,
           ee1_W1, ee1_b1, ee1_W2, ee1_b2,
           ee2_W1, ee2_b1, ee2_W2, ee2_b2,
           ee3_W1, ee3_b1, ee3_W2, ee3_b2):
    Esrc = Esrc.astype(jnp.int32)
    Etgt = Etgt.astype(jnp.int32)
    batch = batch.astype(jnp.int32)

    ef1, ef2, ef3 = _edge_gates(
        edge_features,
        ((ee1_W1, ee1_b1, ee1_W2, ee1_b2),
         (ee2_W1, ee2_b1, ee2_W2, ee2_b2),
         (ee3_W1, ee3_b1, ee3_W2, ee3_b2)))

    s1 = _linear(node_features, gc1_W, gc1_b)
    p1 = _sc_layer(s1, ef1, Esrc, Etgt)
    s2 = _linear_from_partials(p1, gc2_W, gc2_b)
    p2 = _sc_layer(s2, ef2, Esrc, Etgt)
    s3 = _linear_from_partials(p2, gc3_W, gc3_b)
    p3 = _sc_layer(s3, ef3, Esrc, Etgt)
    return _pool(p3, batch)


# speedup vs baseline: 4.9771x; 1.6197x over previous
import functools

import jax
import jax.numpy as jnp
from jax import lax
from jax.experimental import pallas as pl
from jax.experimental.pallas import tpu as pltpu
from jax.experimental.pallas import tpu_sc as plsc

_N = 10000
_E = 320000
_D = 128
_DE = 16
_B = 64

_NC = 2
_NS = 16
_NW = _NC * _NS
_EPW = _E // _NW
_C = 80
_NCH = _EPW // _C
_STRIPE = 624
_TAIL = _N - _NS * _STRIPE
_ZR = 104

_EB = 3200
_NB = 2000

_f32 = jnp.float32


def _gates_body(ef_ref, w11, b11, w12, b12, w21, b21, w22, b22,
                w31, b31, w32, b32, o1, o2, o3):
    ef = ef_ref[...]
    for w1, b1, w2, b2, o in ((w11, b11, w12, b12, o1),
                              (w21, b21, w22, b22, o2),
                              (w31, b31, w32, b32, o3)):
        h = jnp.maximum(
            jnp.dot(ef, w1[...], preferred_element_type=_f32) + b1[...], 0.0)
        o[...] = jax.nn.sigmoid(
            jnp.dot(h, w2[...], preferred_element_type=_f32) + b2[...])


def _edge_gates(ef, params):
    full = lambda arr: pl.BlockSpec(arr.shape, lambda i: (0,) * arr.ndim)
    in_specs = [pl.BlockSpec((_EB, _DE), lambda i: (i, 0))]
    args = [ef]
    for w1, b1, w2, b2 in params:
        b1 = b1.reshape(1, _D)
        b2 = b2.reshape(1, _D)
        for a in (w1, b1, w2, b2):
            in_specs.append(full(a))
            args.append(a)
    out = pl.pallas_call(
        _gates_body,
        grid=(_E // _EB,),
        in_specs=in_specs,
        out_specs=[pl.BlockSpec((_EB, _D), lambda i: (i, 0))] * 3,
        out_shape=[jax.ShapeDtypeStruct((_E, _D), _f32)] * 3,
    )(*args)
    return out


def _lin_body(x_ref, w_ref, b_ref, o_ref):
    o_ref[...] = (jnp.dot(x_ref[...], w_ref[...], preferred_element_type=_f32)
                  + b_ref[...])


def _linear(x, w, b):
    return pl.pallas_call(
        _lin_body,
        grid=(_N // _NB,),
        in_specs=[pl.BlockSpec((_NB, _D), lambda i: (i, 0)),
                  pl.BlockSpec((_D, _D), lambda i: (0, 0)),
                  pl.BlockSpec((1, _D), lambda i: (0, 0))],
        out_specs=pl.BlockSpec((_NB, _D), lambda i: (i, 0)),
        out_shape=jax.ShapeDtypeStruct((_N, _D), _f32),
    )(x, w, b.reshape(1, _D))


def _lin2_body(p_ref, w_ref, b_ref, o_ref):
    x = jnp.maximum(p_ref[0] + p_ref[1], 0.0)
    o_ref[...] = (jnp.dot(x, w_ref[...], preferred_element_type=_f32)
                  + b_ref[...])


def _linear_from_partials(p, w, b):
    return pl.pallas_call(
        _lin2_body,
        grid=(_N // _NB,),
        in_specs=[pl.BlockSpec((2, _NB, _D), lambda i: (0, i, 0)),
                  pl.BlockSpec((_D, _D), lambda i: (0, 0)),
                  pl.BlockSpec((1, _D), lambda i: (0, 0))],
        out_specs=pl.BlockSpec((_NB, _D), lambda i: (i, 0)),
        out_shape=jax.ShapeDtypeStruct((_N, _D), _f32),
    )(p, w, b.reshape(1, _D))


def _pool_body(p_ref, batch_ref, o_ref):
    i = pl.program_id(0)

    @pl.when(i == 0)
    def _():
        o_ref[...] = jnp.zeros_like(o_ref)

    x = p_ref[0] + p_ref[1]
    seg = batch_ref[...]
    gids = lax.broadcasted_iota(jnp.int32, (1, _B), 1)
    onehot = (seg == gids).astype(_f32)
    o_ref[...] += lax.dot_general(
        onehot, x, (((0,), (0,)), ((), ())), preferred_element_type=_f32)


def _pool(p, batch):
    return pl.pallas_call(
        _pool_body,
        grid=(_N // _NB,),
        in_specs=[pl.BlockSpec((2, _NB, _D), lambda i: (0, i, 0)),
                  pl.BlockSpec((_NB, 1), lambda i: (i, 0))],
        out_specs=pl.BlockSpec((_B, _D), lambda i: (0, 0)),
        out_shape=jax.ShapeDtypeStruct((_B, _D), _f32),
    )(p, batch.reshape(_N, 1))


def _sc_body(sup_hbm, ef_hbm, esrc_hbm, etgt_hbm, out_hbm,
             esrc0_v, esrc1_v, etgt0_v, etgt1_v,
             rows0_v, rows1_v, ef0_v, ef1_v, acc_sh,
             si0, si1, st0, st1, sg0, sg1, se0, se1, ss0, ss1):
    c = lax.axis_index("c")
    s = lax.axis_index("s")
    wid = c * _NS + s

    zero16 = jnp.zeros((16,), _f32)

    @pl.loop(0, _C)
    def _zfill(i):
        for j in range(_D // 16):
            rows0_v[i, pl.ds(j * 16, 16)] = zero16

    @pl.loop(0, 7)
    def _zcopy(k):
        pltpu.sync_copy(rows0_v, acc_sh.at[pl.ds(s * _STRIPE + k * _C, _C)])

    pltpu.sync_copy(rows0_v.at[pl.ds(0, _STRIPE - 7 * _C)],
                    acc_sh.at[pl.ds(s * _STRIPE + 7 * _C, _STRIPE - 7 * _C)])

    @pl.when(s == _NS - 1)
    def _ztail():
        pltpu.sync_copy(rows0_v.at[pl.ds(0, _TAIL)],
                        acc_sh.at[pl.ds(_NS * _STRIPE, _TAIL)])

    plsc.subcore_barrier()

    slots = ((esrc0_v, etgt0_v, rows0_v, ef0_v, si0, st0, sg0, se0, ss0),
             (esrc1_v, etgt1_v, rows1_v, ef1_v, si1, st1, sg1, se1, ss1))

    def _esrc_src(i):
        return esrc_hbm.at[pl.ds(wid * _EPW + i * _C, _C)]

    def _etgt_src(i):
        return etgt_hbm.at[pl.ds(wid * _EPW + i * _C, _C)]

    def _ef_src(i):
        return ef_hbm.at[pl.ds(wid * _EPW + i * _C, _C)]

    pltpu.sync_copy(_esrc_src(0), esrc0_v)
    pltpu.sync_copy(_etgt_src(0), etgt0_v)
    pltpu.async_copy(sup_hbm.at[esrc0_v], rows0_v, sg0)
    pltpu.async_copy(_ef_src(0), ef0_v, se0)

    @pl.loop(0, _NCH + 1, step=2)
    def _outer(g):
        for b in range(2):
            i = g + b
            esrc_c, etgt_c, rows_c, ef_c, si_c, st_c, sg_c, se_c, ss_c = slots[b]
            esrc_n, etgt_n, rows_n, ef_n, si_n, st_n, sg_n, se_n, ss_n = slots[1 - b]

            @pl.when(i < _NCH)
            def _iter():
                has_next = i + 1 < _NCH

                @pl.when(jnp.logical_and(has_next, i >= 1))
                def _():
                    pltpu.make_async_copy(
                        rows_n, acc_sh.at[etgt_n], ss_n).wait()

                @pl.when(has_next)
                def _():
                    pltpu.async_copy(_esrc_src(i + 1), esrc_n, si_n)
                    pltpu.async_copy(_etgt_src(i + 1), etgt_n, st_n)

                pltpu.make_async_copy(sup_hbm.at[esrc_c], rows_c, sg_c).wait()

                @pl.when(has_next)
                def _():
                    pltpu.make_async_copy(_esrc_src(i + 1), esrc_n, si_n).wait()
                    pltpu.make_async_copy(_etgt_src(i + 1), etgt_n, st_n).wait()
                    pltpu.async_copy(sup_hbm.at[esrc_n], rows_n, sg_n)
                    pltpu.async_copy(_ef_src(i + 1), ef_n, se_n)

                pltpu.make_async_copy(_ef_src(i), ef_c, se_c).wait()

                @pl.loop(0, _C)
                def _mul(r):
                    for j in range(_D // 16):
                        sl = pl.ds(j * 16, 16)
                        rows_c[r, sl] = rows_c[r, sl] * ef_c[r, sl]

                @pl.when(i >= _NCH - 2)
                def _():
                    pltpu.sync_copy(rows_c, acc_sh.at[etgt_c], add=True)

                @pl.when(i < _NCH - 2)
                def _():
                    pltpu.async_copy(rows_c, acc_sh.at[etgt_c], ss_c,
                                     add=True)

    plsc.subcore_barrier()

    off = s * _STRIPE
    pltpu.sync_copy(acc_sh.at[pl.ds(off, _STRIPE)],
                    out_hbm.at[c, pl.ds(off, _STRIPE)])

    @pl.when(s == _NS - 1)
    def _otail():
        pltpu.sync_copy(acc_sh.at[pl.ds(_NS * _STRIPE, _TAIL)],
                        out_hbm.at[c, pl.ds(_NS * _STRIPE, _TAIL)])


@functools.cache
def _make_sc_layer():
    return pl.kernel(
        _sc_body,
        out_type=jax.ShapeDtypeStruct((_NC, _N, _D), _f32),
        mesh=plsc.VectorSubcoreMesh(core_axis_name="c", subcore_axis_name="s",
                                    num_cores=_NC, num_subcores=_NS),
        scratch_types=[
            pltpu.VMEM((_C,), jnp.int32),
            pltpu.VMEM((_C,), jnp.int32),
            pltpu.VMEM((_C,), jnp.int32),
            pltpu.VMEM((_C,), jnp.int32),
            pltpu.VMEM((_C, _D), _f32),
            pltpu.VMEM((_C, _D), _f32),
            pltpu.VMEM((_C, _D), _f32),
            pltpu.VMEM((_C, _D), _f32),
            pltpu.VMEM_SHARED((_N, _D), _f32),
        ] + [pltpu.SemaphoreType.DMA] * 10,
    )


def _sc_layer(support, ef, esrc, etgt):
    return _make_sc_layer()(support, ef, esrc, etgt)


def kernel(node_features, edge_features, Esrc, Etgt, batch,
           gc1_W, gc1_b, gc2_W, gc2_b, gc3_W, gc3_b,
           ee1_W1, ee1_b1, ee1_W2, ee1_b2,
           ee2_W1, ee2_b1, ee2_W2, ee2_b2,
           ee3_W1, ee3_b1, ee3_W2, ee3_b2):
    Esrc = Esrc.astype(jnp.int32)
    Etgt = Etgt.astype(jnp.int32)
    batch = batch.astype(jnp.int32)

    ef1, ef2, ef3 = _edge_gates(
        edge_features,
        ((ee1_W1, ee1_b1, ee1_W2, ee1_b2),
         (ee2_W1, ee2_b1, ee2_W2, ee2_b2),
         (ee3_W1, ee3_b1, ee3_W2, ee3_b2)))

    s1 = _linear(node_features, gc1_W, gc1_b)
    p1 = _sc_layer(s1, ef1, Esrc, Etgt)
    s2 = _linear_from_partials(p1, gc2_W, gc2_b)
    p2 = _sc_layer(s2, ef2, Esrc, Etgt)
    s3 = _linear_from_partials(p2, gc3_W, gc3_b)
    p3 = _sc_layer(s3, ef3, Esrc, Etgt)
    return _pool(p3, batch)
